# baseline (speedup 1.0000x reference)
import jax
import jax.numpy as jnp
from jax import lax
from jax.experimental import pallas as pl
from jax.experimental.pallas import tpu as pltpu

_INITIAL_VOCAB = 100000
_BLK = 512


def _tc_body(idx_smem, usage_in, ctime_in, emb_in, nemb_in,
             usage_out, ctime_out, emb_out, row_out,
             nemb_v, blk_v, sem_big, sem_small):
    rows, _ = emb_in.shape
    n_chunks = 8
    chunk = rows // n_chunks
    copies = []
    start = 0
    for k in range(n_chunks):
        sz = chunk if k < n_chunks - 1 else rows - start
        cp = pltpu.make_async_copy(
            emb_in.at[pl.ds(start, sz)], emb_out.at[pl.ds(start, sz)], sem_big)
        cp.start()
        copies.append(cp)
        start += sz
    cp_u = pltpu.make_async_copy(usage_in, usage_out, sem_big)
    cp_u.start()
    cp_c = pltpu.make_async_copy(ctime_in, ctime_out, sem_big)
    cp_c.start()

    cp_n = pltpu.make_async_copy(nemb_in, nemb_v, sem_small)
    cp_n.start()
    cp_n.wait()
    cp_r = pltpu.make_async_copy(nemb_v, row_out, sem_small)
    cp_r.start()

    for cp in copies:
        cp.wait()
    cp_u.wait()
    cp_c.wait()
    cp_r.wait()

    tok = idx_smem[0]
    exp_row = tok - _INITIAL_VOCAB

    cp_w = pltpu.make_async_copy(
        nemb_v, emb_out.at[pl.ds(exp_row, 1)], sem_small)
    cp_w.start()

    a = (tok // _BLK) * _BLK
    local = tok - a
    lane = lax.broadcasted_iota(jnp.int32, (_BLK,), 0)
    cp1 = pltpu.make_async_copy(usage_out.at[pl.ds(a, _BLK)], blk_v, sem_small)
    cp1.start()
    cp1.wait()
    blk_v[...] = jnp.where(lane == local, blk_v[...] + 1.0, blk_v[...])
    cp2 = pltpu.make_async_copy(blk_v, usage_out.at[pl.ds(a, _BLK)], sem_small)
    cp2.start()
    cp2.wait()

    cp3 = pltpu.make_async_copy(ctime_out.at[pl.ds(a, _BLK)], blk_v, sem_small)
    cp3.start()
    cp3.wait()
    blk_v[...] = jnp.where(lane == local, jnp.float32(jnp.inf), blk_v[...])
    cp4 = pltpu.make_async_copy(blk_v, ctime_out.at[pl.ds(a, _BLK)], sem_small)
    cp4.start()
    cp4.wait()

    cp_w.wait()


def kernel(token_usage, token_creation_time, expanded_embeddings,
           new_embedding, new_token_id):
    idx = jnp.asarray(new_token_id, jnp.int32).reshape(1)
    usage, ctime, expanded, row = pl.pallas_call(
        _tc_body,
        in_specs=[
            pl.BlockSpec(memory_space=pltpu.SMEM),
            pl.BlockSpec(memory_space=pl.ANY),
            pl.BlockSpec(memory_space=pl.ANY),
            pl.BlockSpec(memory_space=pl.ANY),
            pl.BlockSpec(memory_space=pl.ANY),
        ],
        out_specs=[
            pl.BlockSpec(memory_space=pl.ANY),
            pl.BlockSpec(memory_space=pl.ANY),
            pl.BlockSpec(memory_space=pl.ANY),
            pl.BlockSpec(memory_space=pl.ANY),
        ],
        out_shape=[
            jax.ShapeDtypeStruct(token_usage.shape, token_usage.dtype),
            jax.ShapeDtypeStruct(token_creation_time.shape,
                                 token_creation_time.dtype),
            jax.ShapeDtypeStruct(expanded_embeddings.shape,
                                 expanded_embeddings.dtype),
            jax.ShapeDtypeStruct((1,) + new_embedding.shape,
                                 new_embedding.dtype),
        ],
        scratch_shapes=[
            pltpu.VMEM((1,) + new_embedding.shape, new_embedding.dtype),
            pltpu.VMEM((_BLK,), jnp.float32),
            pltpu.SemaphoreType.DMA,
            pltpu.SemaphoreType.DMA,
        ],
    )(idx, token_usage, token_creation_time, expanded_embeddings,
      new_embedding.reshape(1, -1))
    return (row.reshape(-1), expanded, usage, ctime)


# speedup vs baseline: 15.7677x; 15.7677x over previous
import jax
import jax.numpy as jnp
from jax import lax
from jax.experimental import pallas as pl
from jax.experimental.pallas import tpu as pltpu

_INITIAL_VOCAB = 100000
_GRID = 125


def _body(idx_smem, emb_in, usage_in, ctime_in, nemb,
          emb_out, usage_out, ctime_out, row_out):
    i = pl.program_id(0)
    tok = idx_smem[0]

    eb = emb_in.shape[0]
    exp_row = tok - _INITIAL_VOCAB
    er_local = exp_row - i * eb
    rows = lax.broadcasted_iota(jnp.int32, emb_in.shape, 0)
    emb_out[...] = jnp.where(rows == er_local, nemb[...], emb_in[...])

    _, ub, lanes = usage_in.shape
    ur = tok // lanes - i * ub
    uc = tok % lanes
    r2 = lax.broadcasted_iota(jnp.int32, usage_in.shape, 1)
    c2 = lax.broadcasted_iota(jnp.int32, usage_in.shape, 2)
    hit = (r2 == ur) & (c2 == uc)
    u = usage_in[...]
    usage_out[...] = jnp.where(hit, u + 1.0, u)

    ctime_out[...] = jnp.where(hit, jnp.float32(jnp.inf), ctime_in[...])

    @pl.when(i == 0)
    def _():
        row_out[...] = nemb[...]


def kernel(token_usage, token_creation_time, expanded_embeddings,
           new_embedding, new_token_id):
    idx = jnp.asarray(new_token_id, jnp.int32).reshape(1)
    n_rows, dim = expanded_embeddings.shape
    eb = n_rows // _GRID
    usage2 = token_usage.reshape(_GRID, -1, 64)
    ctime2 = token_creation_time.reshape(_GRID, -1, 64)
    ub = usage2.shape[1]

    expanded, usage, ctime, row = pl.pallas_call(
        _body,
        grid=(_GRID,),
        in_specs=[
            pl.BlockSpec(memory_space=pltpu.SMEM),
            pl.BlockSpec((eb, dim), lambda i: (i, 0)),
            pl.BlockSpec((1, ub, 64), lambda i: (i, 0, 0)),
            pl.BlockSpec((1, ub, 64), lambda i: (i, 0, 0)),
            pl.BlockSpec((1, dim), lambda i: (0, 0)),
        ],
        out_specs=[
            pl.BlockSpec((eb, dim), lambda i: (i, 0)),
            pl.BlockSpec((1, ub, 64), lambda i: (i, 0, 0)),
            pl.BlockSpec((1, ub, 64), lambda i: (i, 0, 0)),
            pl.BlockSpec((1, dim), lambda i: (0, 0)),
        ],
        out_shape=[
            jax.ShapeDtypeStruct(expanded_embeddings.shape, jnp.float32),
            jax.ShapeDtypeStruct(usage2.shape, jnp.float32),
            jax.ShapeDtypeStruct(ctime2.shape, jnp.float32),
            jax.ShapeDtypeStruct((1, dim), jnp.float32),
        ],
    )(idx, expanded_embeddings, usage2, ctime2, new_embedding.reshape(1, -1))
    return (row.reshape(-1), expanded, usage.reshape(-1), ctime.reshape(-1))
